# Initial kernel scaffold; baseline (speedup 1.0000x reference)
import jax
import jax.numpy as jnp
from jax.experimental import pallas as pl


def kernel(x, embed, Wr1, br1, Wr2, Wr2v, Wproj, Wvecmix, Wupd1, bupd1, Wupd2, Wnorm, Wout_vec, Wout_inv, bout_inv):
    raise NotImplementedError("write your pallas kernel here")


# speedup vs baseline: 2.1448x; 2.1448x over previous
import jax
import jax.numpy as jnp
import numpy as np
from jax.experimental import pallas as pl
from jax.experimental.pallas import tpu as pltpu

_B, _N = 128, 64
_C_INV, _C_VEC = 64, 32
_N_RBF, _WIDTH, _L = 8, 64, 2
_N_VEC_OUT, _N_INV_OUT = 32, 64
_E = _N * _N


def _mace_graph_kernel(xs_ref, xr_ref, embed_ref, Wr1_ref, br1_ref, Wr2_ref,
                       Wr2v_ref, Wproj_ref, Wvecmix_ref, Wupd1_ref, bupd1_ref,
                       Wupd2_ref, Wnorm_ref, Wout_vec_ref, Wout_inv_ref,
                       bout_inv_ref, centers_ref,
                       vfx_ref, vfy_ref, vfz_ref, inv_ref):
    f32 = jnp.float32
    xs = xs_ref[0]
    xr = xr_ref[0]

    vec = [xr[:, d:d + 1] - xs[:, d:d + 1] for d in range(3)]
    d2 = vec[0] * vec[0] + vec[1] * vec[1] + vec[2] * vec[2]
    dist = jnp.sqrt(d2)
    inv_dist = 1.0 / (dist + 1e-8)
    unit3 = [(v * inv_dist).reshape(_N, _N, 1) for v in vec]

    e_idx = jax.lax.broadcasted_iota(jnp.int32, (_E, 1), 0)
    mask_flat = jnp.where(e_idx % (_N + 1) == 0, f32(0.0), f32(1.0))

    rbf = jnp.exp(-((dist - centers_ref[...]) ** 2))

    h_inv = jnp.broadcast_to(embed_ref[...], (_N, _C_INV))
    hv = [jnp.zeros((_N, _C_VEC), dtype=f32) for _ in range(3)]
    inv_n = f32(1.0 / _N)

    for l in range(_L):
        hid = jax.nn.silu(
            jnp.dot(rbf, Wr1_ref[l], preferred_element_type=f32) + br1_ref[l])
        R_inv = jnp.dot(hid, Wr2_ref[l], preferred_element_type=f32) * mask_flat
        R_vec = jnp.dot(hid, Wr2v_ref[l], preferred_element_type=f32) * mask_flat
        R_inv3 = R_inv.reshape(_N, _N, _C_INV)
        R_vec3 = R_vec.reshape(_N, _N, _C_VEC)

        agg_inv = jnp.sum(R_inv3 * h_inv[:, None, :], axis=0) * inv_n

        P = jnp.dot(h_inv, Wproj_ref[l], preferred_element_type=f32)
        RP = R_vec3 * P[:, None, :]
        agg_v = []
        for d in range(3):
            t = RP * unit3[d] + R_vec3 * hv[d][:, None, :]
            agg_v.append(jnp.sum(t, axis=0) * inv_n)

        upd = jax.nn.silu(
            jnp.dot(agg_inv, Wupd1_ref[l], preferred_element_type=f32)
            + bupd1_ref[l])
        h_inv = h_inv + jnp.dot(upd, Wupd2_ref[l], preferred_element_type=f32)
        hv = [hv[d] + jnp.dot(agg_v[d], Wvecmix_ref[l], preferred_element_type=f32)
              for d in range(3)]
        vec_norm = jnp.sqrt(hv[0] ** 2 + hv[1] ** 2 + hv[2] ** 2 + 1e-8)
        h_inv = h_inv + jnp.dot(vec_norm, Wnorm_ref[l], preferred_element_type=f32)

    vfx_ref[0] = jnp.dot(hv[0], Wout_vec_ref[...], preferred_element_type=f32)
    vfy_ref[0] = jnp.dot(hv[1], Wout_vec_ref[...], preferred_element_type=f32)
    vfz_ref[0] = jnp.dot(hv[2], Wout_vec_ref[...], preferred_element_type=f32)
    inv_ref[0] = (jnp.dot(h_inv, Wout_inv_ref[...], preferred_element_type=f32)
                  + bout_inv_ref[...])


def _full(shape):
    nd = len(shape)
    return pl.BlockSpec(shape, lambda i, _nd=nd: (0,) * _nd)


def kernel(x, embed, Wr1, br1, Wr2, Wr2v, Wproj, Wvecmix, Wupd1, bupd1, Wupd2,
           Wnorm, Wout_vec, Wout_inv, bout_inv):
    xs = jnp.broadcast_to(x[:, :, None, :], (_B, _N, _N, 3)).reshape(_B, _E, 3)
    xr = jnp.broadcast_to(x[:, None, :, :], (_B, _N, _N, 3)).reshape(_B, _E, 3)
    embed2 = embed.reshape(1, _C_INV)
    br1_3 = br1.reshape(_L, 1, _WIDTH)
    bupd1_3 = bupd1.reshape(_L, 1, _WIDTH)
    bout2 = bout_inv.reshape(1, _N_INV_OUT)
    centers = jnp.asarray(np.linspace(0.0, 3.0, _N_RBF, dtype=np.float32)
                          .reshape(1, _N_RBF))

    out_shapes = (
        jax.ShapeDtypeStruct((_B, _N, _N_VEC_OUT), jnp.float32),
        jax.ShapeDtypeStruct((_B, _N, _N_VEC_OUT), jnp.float32),
        jax.ShapeDtypeStruct((_B, _N, _N_VEC_OUT), jnp.float32),
        jax.ShapeDtypeStruct((_B, _N, _N_INV_OUT), jnp.float32),
    )
    in_specs = [
        pl.BlockSpec((1, _E, 3), lambda i: (i, 0, 0)),
        pl.BlockSpec((1, _E, 3), lambda i: (i, 0, 0)),
        _full((1, _C_INV)),
        _full((_L, _N_RBF, _WIDTH)),
        _full((_L, 1, _WIDTH)),
        _full((_L, _WIDTH, _C_INV)),
        _full((_L, _WIDTH, _C_VEC)),
        _full((_L, _C_INV, _C_VEC)),
        _full((_L, _C_VEC, _C_VEC)),
        _full((_L, _C_INV, _WIDTH)),
        _full((_L, 1, _WIDTH)),
        _full((_L, _WIDTH, _C_INV)),
        _full((_L, _C_VEC, _C_INV)),
        _full((_C_VEC, _N_VEC_OUT)),
        _full((_C_INV, _N_INV_OUT)),
        _full((1, _N_INV_OUT)),
        _full((1, _N_RBF)),
    ]
    out_specs = (
        pl.BlockSpec((1, _N, _N_VEC_OUT), lambda i: (i, 0, 0)),
        pl.BlockSpec((1, _N, _N_VEC_OUT), lambda i: (i, 0, 0)),
        pl.BlockSpec((1, _N, _N_VEC_OUT), lambda i: (i, 0, 0)),
        pl.BlockSpec((1, _N, _N_INV_OUT), lambda i: (i, 0, 0)),
    )

    vfx, vfy, vfz, inv = pl.pallas_call(
        _mace_graph_kernel,
        grid=(_B,),
        in_specs=in_specs,
        out_specs=out_specs,
        out_shape=out_shapes,
        compiler_params=pltpu.CompilerParams(
            dimension_semantics=("arbitrary",)),
    )(xs, xr, embed2, Wr1, br1_3, Wr2, Wr2v, Wproj, Wvecmix, Wupd1, bupd1_3,
      Wupd2, Wnorm, Wout_vec, Wout_inv, bout2, centers)

    vector_features = jnp.stack([vfx, vfy, vfz], axis=-1)
    return vector_features, inv


# speedup vs baseline: 3.3471x; 1.5606x over previous
import jax
import jax.numpy as jnp
import numpy as np
from jax.experimental import pallas as pl
from jax.experimental.pallas import tpu as pltpu

_B, _N = 128, 64
_C_INV, _C_VEC = 64, 32
_N_RBF, _WIDTH, _L = 8, 64, 2
_N_VEC_OUT, _N_INV_OUT = 32, 64
_E = _N * _N


def _mace_graph_kernel(xs_ref, xr_ref, embed_ref, Wr1_ref, br1_ref, Wr2_ref,
                       Wr2v_ref, Wproj_ref, Wvecmix_ref, Wupd1_ref, bupd1_ref,
                       Wupd2_ref, Wnorm_ref, Wout_vec_ref, Wout_inv_ref,
                       bout_inv_ref, centers_ref,
                       vfx_ref, vfy_ref, vfz_ref, inv_ref):
    f32 = jnp.float32
    xs = xs_ref[0]
    xr = xr_ref[0]

    vec_all = xr - xs
    d2 = jnp.sum(vec_all * vec_all, axis=1, keepdims=True)
    dist = jnp.sqrt(d2)
    inv_dist = 1.0 / (dist + 1e-8)
    unit_all = vec_all * inv_dist
    d_iota = jax.lax.broadcasted_iota(jnp.int32, (3, 3 * _C_VEC), 0)
    k_iota = jax.lax.broadcasted_iota(jnp.int32, (3, 3 * _C_VEC), 1)
    sel3 = jnp.where(k_iota // _C_VEC == d_iota, f32(1.0), f32(0.0))
    ub = jnp.dot(unit_all, sel3, preferred_element_type=f32)
    ub3 = [ub[:, d * _C_VEC:(d + 1) * _C_VEC].reshape(_N, _N, _C_VEC)
           for d in range(3)]

    rbf = jnp.exp(-((dist - centers_ref[...]) ** 2))
    rbf0 = jnp.exp(-(centers_ref[...] ** 2))

    embed_row = embed_ref[...]
    inv_n = f32(1.0 / _N)

    for l in range(_L):
        hid = jax.nn.silu(
            jnp.dot(rbf, Wr1_ref[l], preferred_element_type=f32) + br1_ref[l])
        hid0 = jax.nn.silu(
            jnp.dot(rbf0, Wr1_ref[l], preferred_element_type=f32) + br1_ref[l])
        R_inv = jnp.dot(hid, Wr2_ref[l], preferred_element_type=f32)
        R_vec = jnp.dot(hid, Wr2v_ref[l], preferred_element_type=f32)
        Rd_inv = jnp.dot(hid0, Wr2_ref[l], preferred_element_type=f32)

        R_vec3 = R_vec.reshape(_N, _N, _C_VEC)
        if l == 0:
            agg_inv = (jnp.sum(R_inv.reshape(_N, _N, _C_INV), axis=0)
                       - Rd_inv) * (embed_row * inv_n)
            P0 = jnp.dot(embed_row, Wproj_ref[l], preferred_element_type=f32)
            agg_v = [jnp.sum(R_vec3 * ub3[d], axis=0) * (P0 * inv_n)
                     for d in range(3)]
            h_base = embed_row
        else:
            Rd_vec = jnp.dot(hid0, Wr2v_ref[l], preferred_element_type=f32)
            agg_inv = (jnp.sum(R_inv.reshape(_N, _N, _C_INV)
                               * h_inv[:, None, :], axis=0)
                       - Rd_inv * h_inv) * inv_n
            P = jnp.dot(h_inv, Wproj_ref[l], preferred_element_type=f32)
            RP = R_vec3 * P[:, None, :]
            agg_v = []
            for d in range(3):
                t = RP * ub3[d] + R_vec3 * hv[d][:, None, :]
                agg_v.append((jnp.sum(t, axis=0) - Rd_vec * hv[d]) * inv_n)
            h_base = h_inv

        upd = jax.nn.silu(
            jnp.dot(agg_inv, Wupd1_ref[l], preferred_element_type=f32)
            + bupd1_ref[l])
        h_inv = h_base + jnp.dot(upd, Wupd2_ref[l], preferred_element_type=f32)
        if l == 0:
            hv = [jnp.dot(agg_v[d], Wvecmix_ref[l], preferred_element_type=f32)
                  for d in range(3)]
        else:
            hv = [hv[d] + jnp.dot(agg_v[d], Wvecmix_ref[l],
                                  preferred_element_type=f32)
                  for d in range(3)]
        vec_norm = jnp.sqrt(hv[0] ** 2 + hv[1] ** 2 + hv[2] ** 2 + 1e-8)
        h_inv = h_inv + jnp.dot(vec_norm, Wnorm_ref[l], preferred_element_type=f32)

    vfx_ref[0] = jnp.dot(hv[0], Wout_vec_ref[...], preferred_element_type=f32)
    vfy_ref[0] = jnp.dot(hv[1], Wout_vec_ref[...], preferred_element_type=f32)
    vfz_ref[0] = jnp.dot(hv[2], Wout_vec_ref[...], preferred_element_type=f32)
    inv_ref[0] = (jnp.dot(h_inv, Wout_inv_ref[...], preferred_element_type=f32)
                  + bout_inv_ref[...])


def _full(shape):
    nd = len(shape)
    return pl.BlockSpec(shape, lambda i, _nd=nd: (0,) * _nd)


def kernel(x, embed, Wr1, br1, Wr2, Wr2v, Wproj, Wvecmix, Wupd1, bupd1, Wupd2,
           Wnorm, Wout_vec, Wout_inv, bout_inv):
    xs = jnp.broadcast_to(x[:, :, None, :], (_B, _N, _N, 3)).reshape(_B, _E, 3)
    xr = jnp.broadcast_to(x[:, None, :, :], (_B, _N, _N, 3)).reshape(_B, _E, 3)
    embed2 = embed.reshape(1, _C_INV)
    br1_3 = br1.reshape(_L, 1, _WIDTH)
    bupd1_3 = bupd1.reshape(_L, 1, _WIDTH)
    bout2 = bout_inv.reshape(1, _N_INV_OUT)
    centers = jnp.asarray(np.linspace(0.0, 3.0, _N_RBF, dtype=np.float32)
                          .reshape(1, _N_RBF))

    out_shapes = (
        jax.ShapeDtypeStruct((_B, _N, _N_VEC_OUT), jnp.float32),
        jax.ShapeDtypeStruct((_B, _N, _N_VEC_OUT), jnp.float32),
        jax.ShapeDtypeStruct((_B, _N, _N_VEC_OUT), jnp.float32),
        jax.ShapeDtypeStruct((_B, _N, _N_INV_OUT), jnp.float32),
    )
    in_specs = [
        pl.BlockSpec((1, _E, 3), lambda i: (i, 0, 0)),
        pl.BlockSpec((1, _E, 3), lambda i: (i, 0, 0)),
        _full((1, _C_INV)),
        _full((_L, _N_RBF, _WIDTH)),
        _full((_L, 1, _WIDTH)),
        _full((_L, _WIDTH, _C_INV)),
        _full((_L, _WIDTH, _C_VEC)),
        _full((_L, _C_INV, _C_VEC)),
        _full((_L, _C_VEC, _C_VEC)),
        _full((_L, _C_INV, _WIDTH)),
        _full((_L, 1, _WIDTH)),
        _full((_L, _WIDTH, _C_INV)),
        _full((_L, _C_VEC, _C_INV)),
        _full((_C_VEC, _N_VEC_OUT)),
        _full((_C_INV, _N_INV_OUT)),
        _full((1, _N_INV_OUT)),
        _full((1, _N_RBF)),
    ]
    out_specs = (
        pl.BlockSpec((1, _N, _N_VEC_OUT), lambda i: (i, 0, 0)),
        pl.BlockSpec((1, _N, _N_VEC_OUT), lambda i: (i, 0, 0)),
        pl.BlockSpec((1, _N, _N_VEC_OUT), lambda i: (i, 0, 0)),
        pl.BlockSpec((1, _N, _N_INV_OUT), lambda i: (i, 0, 0)),
    )

    vfx, vfy, vfz, inv = pl.pallas_call(
        _mace_graph_kernel,
        grid=(_B,),
        in_specs=in_specs,
        out_specs=out_specs,
        out_shape=out_shapes,
        compiler_params=pltpu.CompilerParams(
            dimension_semantics=("arbitrary",)),
    )(xs, xr, embed2, Wr1, br1_3, Wr2, Wr2v, Wproj, Wvecmix, Wupd1, bupd1_3,
      Wupd2, Wnorm, Wout_vec, Wout_inv, bout2, centers)

    vector_features = jnp.stack([vfx, vfy, vfz], axis=-1)
    return vector_features, inv


# speedup vs baseline: 3.4704x; 1.0368x over previous
import jax
import jax.numpy as jnp
import numpy as np
from jax.experimental import pallas as pl
from jax.experimental.pallas import tpu as pltpu

_B, _N = 128, 64
_C_INV, _C_VEC = 64, 32
_N_RBF, _WIDTH, _L = 8, 64, 2
_N_VEC_OUT, _N_INV_OUT = 32, 64
_E = _N * _N
_G = 2
_GE = _G * _E
_GN = _G * _N


def _mace_graph_kernel(xs_ref, xr_ref, embed_ref, Wr1_ref, br1_ref, Wr2_ref,
                       Wr2v_ref, Wproj_ref, Wvecmix_ref, Wupd1_ref, bupd1_ref,
                       Wupd2_ref, Wnorm_ref, Wout_vec_ref, Wout_inv_ref,
                       bout_inv_ref, centers_ref,
                       vfx_ref, vfy_ref, vfz_ref, inv_ref):
    f32 = jnp.float32
    xs = xs_ref[...].reshape(_GE, 3)
    xr = xr_ref[...].reshape(_GE, 3)

    vec_all = xr - xs
    d2 = jnp.sum(vec_all * vec_all, axis=1, keepdims=True)
    dist = jnp.sqrt(d2)
    inv_dist = 1.0 / (dist + 1e-8)
    unit_all = vec_all * inv_dist
    d_iota = jax.lax.broadcasted_iota(jnp.int32, (3, 3 * _C_VEC), 0)
    k_iota = jax.lax.broadcasted_iota(jnp.int32, (3, 3 * _C_VEC), 1)
    sel3 = jnp.where(k_iota // _C_VEC == d_iota, f32(1.0), f32(0.0))
    ub = jnp.dot(unit_all, sel3, preferred_element_type=f32)
    ub4 = [ub[:, d * _C_VEC:(d + 1) * _C_VEC].reshape(_G, _N, _N, _C_VEC)
           for d in range(3)]

    rbf = jnp.exp(-((dist - centers_ref[...]) ** 2))
    rbf0 = jnp.exp(-(centers_ref[...] ** 2))

    embed_row = embed_ref[...]
    inv_n = f32(1.0 / _N)

    for l in range(_L):
        hid = jax.nn.silu(
            jnp.dot(rbf, Wr1_ref[l], preferred_element_type=f32) + br1_ref[l])
        hid0 = jax.nn.silu(
            jnp.dot(rbf0, Wr1_ref[l], preferred_element_type=f32) + br1_ref[l])
        R_inv = jnp.dot(hid, Wr2_ref[l], preferred_element_type=f32)
        R_vec = jnp.dot(hid, Wr2v_ref[l], preferred_element_type=f32)
        Rd_inv = jnp.dot(hid0, Wr2_ref[l], preferred_element_type=f32)

        R_vec4 = R_vec.reshape(_G, _N, _N, _C_VEC)
        if l == 0:
            agg_inv = ((jnp.sum(R_inv.reshape(_G, _N, _N, _C_INV), axis=1)
                        .reshape(_GN, _C_INV) - Rd_inv)
                       * (embed_row * inv_n))
            P0 = jnp.dot(embed_row, Wproj_ref[l], preferred_element_type=f32)
            agg_v = [jnp.sum(R_vec4 * ub4[d], axis=1).reshape(_GN, _C_VEC)
                     * (P0 * inv_n)
                     for d in range(3)]
            h_base = embed_row
        else:
            Rd_vec = jnp.dot(hid0, Wr2v_ref[l], preferred_element_type=f32)
            h4 = h_inv.reshape(_G, _N, 1, _C_INV)
            agg_inv = ((jnp.sum(R_inv.reshape(_G, _N, _N, _C_INV) * h4, axis=1)
                        .reshape(_GN, _C_INV) - Rd_inv * h_inv) * inv_n)
            P = jnp.dot(h_inv, Wproj_ref[l], preferred_element_type=f32)
            RP = R_vec4 * P.reshape(_G, _N, 1, _C_VEC)
            agg_v = []
            for d in range(3):
                t = RP * ub4[d] + R_vec4 * hv[d].reshape(_G, _N, 1, _C_VEC)
                agg_v.append((jnp.sum(t, axis=1).reshape(_GN, _C_VEC)
                              - Rd_vec * hv[d]) * inv_n)
            h_base = h_inv

        upd = jax.nn.silu(
            jnp.dot(agg_inv, Wupd1_ref[l], preferred_element_type=f32)
            + bupd1_ref[l])
        h_inv = h_base + jnp.dot(upd, Wupd2_ref[l], preferred_element_type=f32)
        if l == 0:
            hv = [jnp.dot(agg_v[d], Wvecmix_ref[l], preferred_element_type=f32)
                  for d in range(3)]
        else:
            hv = [hv[d] + jnp.dot(agg_v[d], Wvecmix_ref[l],
                                  preferred_element_type=f32)
                  for d in range(3)]
        vec_norm = jnp.sqrt(hv[0] ** 2 + hv[1] ** 2 + hv[2] ** 2 + 1e-8)
        h_inv = h_inv + jnp.dot(vec_norm, Wnorm_ref[l], preferred_element_type=f32)

    vfx_ref[...] = jnp.dot(hv[0], Wout_vec_ref[...],
                           preferred_element_type=f32).reshape(_G, _N, _N_VEC_OUT)
    vfy_ref[...] = jnp.dot(hv[1], Wout_vec_ref[...],
                           preferred_element_type=f32).reshape(_G, _N, _N_VEC_OUT)
    vfz_ref[...] = jnp.dot(hv[2], Wout_vec_ref[...],
                           preferred_element_type=f32).reshape(_G, _N, _N_VEC_OUT)
    inv_ref[...] = (jnp.dot(h_inv, Wout_inv_ref[...], preferred_element_type=f32)
                    + bout_inv_ref[...]).reshape(_G, _N, _N_INV_OUT)


def _full(shape):
    nd = len(shape)
    return pl.BlockSpec(shape, lambda i, _nd=nd: (0,) * _nd)


def kernel(x, embed, Wr1, br1, Wr2, Wr2v, Wproj, Wvecmix, Wupd1, bupd1, Wupd2,
           Wnorm, Wout_vec, Wout_inv, bout_inv):
    xs = jnp.broadcast_to(x[:, :, None, :], (_B, _N, _N, 3)).reshape(_B, _E, 3)
    xr = jnp.broadcast_to(x[:, None, :, :], (_B, _N, _N, 3)).reshape(_B, _E, 3)
    embed2 = embed.reshape(1, _C_INV)
    br1_3 = br1.reshape(_L, 1, _WIDTH)
    bupd1_3 = bupd1.reshape(_L, 1, _WIDTH)
    bout2 = bout_inv.reshape(1, _N_INV_OUT)
    centers = jnp.asarray(np.linspace(0.0, 3.0, _N_RBF, dtype=np.float32)
                          .reshape(1, _N_RBF))

    out_shapes = (
        jax.ShapeDtypeStruct((_B, _N, _N_VEC_OUT), jnp.float32),
        jax.ShapeDtypeStruct((_B, _N, _N_VEC_OUT), jnp.float32),
        jax.ShapeDtypeStruct((_B, _N, _N_VEC_OUT), jnp.float32),
        jax.ShapeDtypeStruct((_B, _N, _N_INV_OUT), jnp.float32),
    )
    in_specs = [
        pl.BlockSpec((_G, _E, 3), lambda i: (i, 0, 0)),
        pl.BlockSpec((_G, _E, 3), lambda i: (i, 0, 0)),
        _full((1, _C_INV)),
        _full((_L, _N_RBF, _WIDTH)),
        _full((_L, 1, _WIDTH)),
        _full((_L, _WIDTH, _C_INV)),
        _full((_L, _WIDTH, _C_VEC)),
        _full((_L, _C_INV, _C_VEC)),
        _full((_L, _C_VEC, _C_VEC)),
        _full((_L, _C_INV, _WIDTH)),
        _full((_L, 1, _WIDTH)),
        _full((_L, _WIDTH, _C_INV)),
        _full((_L, _C_VEC, _C_INV)),
        _full((_C_VEC, _N_VEC_OUT)),
        _full((_C_INV, _N_INV_OUT)),
        _full((1, _N_INV_OUT)),
        _full((1, _N_RBF)),
    ]
    out_specs = (
        pl.BlockSpec((_G, _N, _N_VEC_OUT), lambda i: (i, 0, 0)),
        pl.BlockSpec((_G, _N, _N_VEC_OUT), lambda i: (i, 0, 0)),
        pl.BlockSpec((_G, _N, _N_VEC_OUT), lambda i: (i, 0, 0)),
        pl.BlockSpec((_G, _N, _N_INV_OUT), lambda i: (i, 0, 0)),
    )

    vfx, vfy, vfz, inv = pl.pallas_call(
        _mace_graph_kernel,
        grid=(_B // _G,),
        in_specs=in_specs,
        out_specs=out_specs,
        out_shape=out_shapes,
        compiler_params=pltpu.CompilerParams(
            dimension_semantics=("arbitrary",)),
    )(xs, xr, embed2, Wr1, br1_3, Wr2, Wr2v, Wproj, Wvecmix, Wupd1, bupd1_3,
      Wupd2, Wnorm, Wout_vec, Wout_inv, bout2, centers)

    vector_features = jnp.stack([vfx, vfy, vfz], axis=-1)
    return vector_features, inv
